# baseline (speedup 1.0000x reference)
import functools

import jax
import jax.numpy as jnp
from jax import lax
from jax.experimental import pallas as pl
from jax.experimental.pallas import tpu as pltpu
from jax.experimental.pallas import tpu_sc as plsc

EMBED = 768
HEADS = 12
HDIM = EMBED // HEADS
DEPTH = 12
MLPR = 4
PATCH = 16
IMG = 224
NPATCH = (IMG // PATCH) ** 2
NUM_CLASSES = 100
NUM_ADAPT = 6
NUM_FIXED = DEPTH - NUM_ADAPT
MAX_BUDGET = 100
NPAD = 200

_PREC = lax.Precision.DEFAULT


def _ln(x, w, b, eps):
    m = jnp.mean(x, axis=-1, keepdims=True)
    v = jnp.mean((x - m) ** 2, axis=-1, keepdims=True)
    return (x - m) * lax.rsqrt(v + eps) * w + b


def _gelu(x):
    return 0.5 * x * (1.0 + lax.erf(x * (2.0 ** -0.5)))


def _block_compute(lo, hi, x_ref, ln1w, ln1b, qkvw, qkvb, projw, projb,
                   ln2w, ln2b, fc1w, fc1b, fc2w, fc2b, o_ref):
    x = x_ref[0]
    h = _ln(x, ln1w[0], ln1b[0], 1e-6)
    qkv = lax.dot_general(h, qkvw[...], (((1,), (1,)), ((), ())),
                          preferred_element_type=jnp.float32,
                          precision=_PREC) + qkvb[0]
    scale = HDIM ** -0.5
    kio = lax.broadcasted_iota(jnp.int32, (NPAD, NPAD), 1)
    kmask = (kio >= lo) & (kio < hi)
    ss = []
    for hh in range(HEADS):
        q = qkv[:, hh * HDIM:(hh + 1) * HDIM]
        k = qkv[:, EMBED + hh * HDIM:EMBED + (hh + 1) * HDIM]
        ss.append(lax.dot_general(q * scale, k, (((1,), (1,)), ((), ())),
                                  preferred_element_type=jnp.float32,
                                  precision=_PREC))
    ps = []
    for hh in range(HEADS):
        e = jnp.where(kmask, jnp.exp(ss[hh]), 0.0)
        ps.append(e * (1.0 / jnp.sum(e, axis=-1, keepdims=True)))
    outs = []
    for hh in range(HEADS):
        v = qkv[:, 2 * EMBED + hh * HDIM:2 * EMBED + (hh + 1) * HDIM]
        outs.append(lax.dot_general(ps[hh], v, (((1,), (0,)), ((), ())),
                                    preferred_element_type=jnp.float32,
                                    precision=_PREC))
    att = jnp.concatenate(outs, axis=1)
    x = x + lax.dot_general(att, projw[...], (((1,), (1,)), ((), ())),
                            preferred_element_type=jnp.float32,
                            precision=_PREC) + projb[0]
    h = _ln(x, ln2w[0], ln2b[0], 1e-6)
    h = _gelu(lax.dot_general(h, fc1w[...], (((1,), (1,)), ((), ())),
                              preferred_element_type=jnp.float32,
                              precision=_PREC) + fc1b[0])
    x = x + lax.dot_general(h, fc2w[...], (((1,), (1,)), ((), ())),
                            preferred_element_type=jnp.float32,
                            precision=_PREC) + fc2b[0]
    o_ref[0] = x


def _block_kernel_plain(lo, hi, x_ref, *refs):
    _block_compute(lo, hi, x_ref, *refs)


def _block_compute2(lo, hi, x_ref, ln1w, ln1b, qkvw, qkvb, projw, projb,
                    ln2w, ln2b, fc1w, fc1b, fc2w, fc2b, o_ref):
    x = x_ref[...].reshape(4 * NPAD, EMBED)
    h = _ln(x, ln1w[0], ln1b[0], 1e-6)
    qkv = lax.dot_general(h, qkvw[...], (((1,), (1,)), ((), ())),
                          preferred_element_type=jnp.float32,
                          precision=_PREC) + qkvb[0]
    scale = HDIM ** -0.5
    kio = lax.broadcasted_iota(jnp.int32, (NPAD, NPAD), 1)
    kmask = (kio >= lo) & (kio < hi)
    atts = []
    for s in range(4):
        base = s * NPAD
        ss = []
        for hh in range(HEADS):
            q = qkv[base:base + NPAD, hh * HDIM:(hh + 1) * HDIM]
            k = qkv[base:base + NPAD, EMBED + hh * HDIM:EMBED + (hh + 1) * HDIM]
            ss.append(lax.dot_general(q * scale, k, (((1,), (1,)), ((), ())),
                                      preferred_element_type=jnp.float32,
                                      precision=_PREC))
        ps = []
        for hh in range(HEADS):
            e = jnp.where(kmask, jnp.exp(ss[hh]), 0.0)
            ps.append(e * (1.0 / jnp.sum(e, axis=-1, keepdims=True)))
        outs = []
        for hh in range(HEADS):
            v = qkv[base:base + NPAD,
                    2 * EMBED + hh * HDIM:2 * EMBED + (hh + 1) * HDIM]
            outs.append(lax.dot_general(ps[hh], v, (((1,), (0,)), ((), ())),
                                        preferred_element_type=jnp.float32,
                                        precision=_PREC))
        atts.append(jnp.concatenate(outs, axis=1))
    att = jnp.concatenate(atts, axis=0)
    x = x + lax.dot_general(att, projw[...], (((1,), (1,)), ((), ())),
                            preferred_element_type=jnp.float32,
                            precision=_PREC) + projb[0]
    h = _ln(x, ln2w[0], ln2b[0], 1e-6)
    h = _gelu(lax.dot_general(h, fc1w[...], (((1,), (1,)), ((), ())),
                              preferred_element_type=jnp.float32,
                              precision=_PREC) + fc1b[0])
    x = x + lax.dot_general(h, fc2w[...], (((1,), (1,)), ((), ())),
                            preferred_element_type=jnp.float32,
                            precision=_PREC) + fc2b[0]
    o_ref[...] = x.reshape(4, NPAD, EMBED)


def _block_kernel_gated(lo, hi, active_ref, x_ref, *refs):
    b = pl.program_id(0)
    o_ref = refs[-1]

    @pl.when(active_ref[b] != 0)
    def _():
        _block_compute(lo, hi, x_ref, *refs)

    @pl.when(active_ref[b] == 0)
    def _():
        o_ref[0] = x_ref[0]


def _bcast8(v):
    return jnp.broadcast_to(v[None, :], (8, v.shape[0]))


def _run_block(xx, blk, lo, hi, active=None):
    bsz = xx.shape[0]
    wparams = [
        _bcast8(blk['ln1_w']), _bcast8(blk['ln1_b']),
        blk['qkv_w'], _bcast8(blk['qkv_b']),
        blk['proj_w'], _bcast8(blk['proj_b']),
        _bcast8(blk['ln2_w']), _bcast8(blk['ln2_b']),
        blk['fc1_w'], _bcast8(blk['fc1_b']),
        blk['fc2_w'], _bcast8(blk['fc2_b']),
    ]

    def wspec(arr):
        nd = arr.ndim
        return pl.BlockSpec(arr.shape, lambda b, _n=nd: (0,) * _n)

    x_spec = pl.BlockSpec((1, NPAD, EMBED), lambda b: (b, 0, 0))
    w_specs = [wspec(a) for a in wparams]
    out_spec = pl.BlockSpec((1, NPAD, EMBED), lambda b: (b, 0, 0))
    out_shape = jax.ShapeDtypeStruct((bsz, NPAD, EMBED), jnp.float32)

    if active is None:
        x2_spec = pl.BlockSpec((4, NPAD, EMBED), lambda b: (b, 0, 0))
        return pl.pallas_call(
            functools.partial(_block_compute2, lo, hi),
            grid=(bsz // 4,),
            in_specs=[x2_spec] + w_specs,
            out_specs=x2_spec,
            out_shape=out_shape,
        )(xx, *wparams)
    act_spec = pl.BlockSpec(memory_space=pltpu.SMEM)
    return pl.pallas_call(
        functools.partial(_block_kernel_gated, lo, hi),
        grid=(bsz,),
        in_specs=[act_spec, x_spec] + w_specs,
        out_specs=out_spec,
        out_shape=out_shape,
    )(active, xx, *wparams)


def _patch_kernel(p_ref, w_ref, b_ref, o_ref):
    o_ref[...] = lax.dot_general(
        p_ref[...], w_ref[...], (((1,), (1,)), ((), ())),
        preferred_element_type=jnp.float32, precision=_PREC) + b_ref[0]


def _patch_embed(x, patch_w, patch_b):
    bsz = x.shape[0]
    g = IMG // PATCH
    pch = x.reshape(bsz, 3, g, PATCH, g, PATCH)
    pch = pch.transpose(0, 2, 4, 1, 3, 5).reshape(bsz * NPATCH, 3 * PATCH * PATCH)
    wm = patch_w.reshape(EMBED, -1)
    bb = _bcast8(patch_b)
    rows = bsz * NPATCH
    return pl.pallas_call(
        _patch_kernel,
        grid=(1,),
        in_specs=[
            pl.BlockSpec((rows, 3 * PATCH * PATCH), lambda i: (0, 0)),
            pl.BlockSpec(wm.shape, lambda i: (0, 0)),
            pl.BlockSpec(bb.shape, lambda i: (0, 0)),
        ],
        out_specs=pl.BlockSpec((rows, EMBED), lambda i: (0, 0)),
        out_shape=jax.ShapeDtypeStruct((rows, EMBED), jnp.float32),
    )(pch, wm, bb).reshape(bsz, NPATCH, EMBED)


def _gate_kernel_sc(logits_hbm, budget_hbm, out_hbm, lg_v, bud_v, act_v):
    wid = lax.axis_index("s") * 2 + lax.axis_index("c")

    @pl.when(wid == 0)
    def _():
        pltpu.sync_copy(logits_hbm, lg_v)
        pltpu.sync_copy(budget_hbm, bud_v)
        b6 = bud_v[...] * float(NUM_ADAPT)
        t = b6.astype(jnp.int32)
        k = t + jnp.where(t.astype(jnp.float32) < b6, 1, 0)
        k = jnp.maximum(k, 1)
        ls = [lg_v[j] for j in range(NUM_ADAPT)]
        one = jnp.full((16,), 1, jnp.int32)
        zero = jnp.full((16,), 0, jnp.int32)
        for j in range(NUM_ADAPT):
            r = zero
            for j2 in range(NUM_ADAPT):
                if j2 == j:
                    continue
                beats = ls[j2] > ls[j]
                if j2 < j:
                    beats = beats | (ls[j2] == ls[j])
                r = r + jnp.where(beats, one, zero)
            act_v[j] = jnp.where(r < k, one, zero)
        pltpu.sync_copy(act_v, out_hbm)


def _gate_sc(logits, budget):
    mesh = plsc.VectorSubcoreMesh(core_axis_name="c", subcore_axis_name="s")
    bsz = budget.shape[0]
    f = functools.partial(
        pl.kernel, mesh=mesh,
        out_type=jax.ShapeDtypeStruct((NUM_ADAPT, bsz), jnp.int32),
        scratch_types=[
            pltpu.VMEM((NUM_ADAPT, bsz), jnp.float32),
            pltpu.VMEM((bsz,), jnp.float32),
            pltpu.VMEM((NUM_ADAPT, bsz), jnp.int32),
        ],
    )(_gate_kernel_sc)
    return f(logits.T, budget)


def kernel(x, budget, params, lat_table):
    bsz = x.shape[0]
    b2 = budget[:, None]

    scaled = b2 * (MAX_BUDGET - 1)
    idx = jnp.round(scaled).astype(jnp.int32).squeeze(-1)
    enc = jnp.take(lat_table, idx, axis=0)
    h = enc @ params['lat_fc1_w'].T + params['lat_fc1_b']
    h = _gelu(h)
    h = _ln(h, params['lat_ln_w'], params['lat_ln_b'], 1e-5)
    lat_tok = (h @ params['lat_fc2_w'].T + params['lat_fc2_b'])[:, None, :]

    tokens = _patch_embed(x, params['patch_w'], params['patch_b'])

    cls = jnp.broadcast_to(params['cls_token'], (bsz, 1, EMBED))
    n_fixed_valid = NPATCH + 2
    xx = jnp.concatenate(
        [jnp.concatenate([lat_tok, cls, tokens], axis=1) + params['pos_embed'],
         jnp.zeros((bsz, NPAD - n_fixed_valid, EMBED), jnp.float32)], axis=1)

    for i in range(NUM_FIXED):
        xx = _run_block(xx, params['blocks'][i], 0, n_fixed_valid)

    lat_repr = xx[:, 0]
    logits = lat_repr @ params['sched_w'].T + params['sched_b']
    active_t = _gate_sc(logits, budget)

    toks = xx
    for i in range(NUM_ADAPT):
        blk = params['blocks'][NUM_FIXED + i]
        toks = _run_block(toks, blk, 1, n_fixed_valid, active=active_t[i])

    cls_out = _ln(toks[:, 1], params['norm_w'], params['norm_b'], 1e-6)
    return cls_out @ params['head_w'].T + params['head_b']


# speedup vs baseline: 1.0187x; 1.0187x over previous
import functools

import jax
import jax.numpy as jnp
from jax import lax
from jax.experimental import pallas as pl
from jax.experimental.pallas import tpu as pltpu
from jax.experimental.pallas import tpu_sc as plsc

EMBED = 768
HEADS = 12
HDIM = EMBED // HEADS
DEPTH = 12
MLPR = 4
PATCH = 16
IMG = 224
NPATCH = (IMG // PATCH) ** 2
NUM_CLASSES = 100
NUM_ADAPT = 6
NUM_FIXED = DEPTH - NUM_ADAPT
MAX_BUDGET = 100
NPAD = 200

_PREC = lax.Precision.DEFAULT


def _ln(x, w, b, eps):
    m = jnp.mean(x, axis=-1, keepdims=True)
    v = jnp.mean((x - m) ** 2, axis=-1, keepdims=True)
    return (x - m) * lax.rsqrt(v + eps) * w + b


def _gelu(x):
    return 0.5 * x * (1.0 + lax.erf(x * (2.0 ** -0.5)))


def _block_compute(lo, hi, x_ref, ln1w, ln1b, qkvw, qkvb, projw, projb,
                   ln2w, ln2b, fc1w, fc1b, fc2w, fc2b, o_ref):
    x = x_ref[0]
    h = _ln(x, ln1w[0], ln1b[0], 1e-6)
    qkv = lax.dot_general(h, qkvw[...], (((1,), (1,)), ((), ())),
                          preferred_element_type=jnp.float32,
                          precision=_PREC) + qkvb[0]
    scale = HDIM ** -0.5
    kio = lax.broadcasted_iota(jnp.int32, (NPAD, NPAD), 1)
    kmask = (kio >= lo) & (kio < hi)
    ss = []
    for hh in range(HEADS):
        q = qkv[:, hh * HDIM:(hh + 1) * HDIM]
        k = qkv[:, EMBED + hh * HDIM:EMBED + (hh + 1) * HDIM]
        ss.append(lax.dot_general(q * scale, k, (((1,), (1,)), ((), ())),
                                  preferred_element_type=jnp.float32,
                                  precision=_PREC))
    ps = []
    for hh in range(HEADS):
        e = jnp.where(kmask, jnp.exp(ss[hh]), 0.0)
        ps.append(e * (1.0 / jnp.sum(e, axis=-1, keepdims=True)))
    outs = []
    for hh in range(HEADS):
        v = qkv[:, 2 * EMBED + hh * HDIM:2 * EMBED + (hh + 1) * HDIM]
        outs.append(lax.dot_general(ps[hh], v, (((1,), (0,)), ((), ())),
                                    preferred_element_type=jnp.float32,
                                    precision=_PREC))
    att = jnp.concatenate(outs, axis=1)
    x = x + lax.dot_general(att, projw[...], (((1,), (1,)), ((), ())),
                            preferred_element_type=jnp.float32,
                            precision=_PREC) + projb[0]
    h = _ln(x, ln2w[0], ln2b[0], 1e-6)
    h = _gelu(lax.dot_general(h, fc1w[...], (((1,), (1,)), ((), ())),
                              preferred_element_type=jnp.float32,
                              precision=_PREC) + fc1b[0])
    x = x + lax.dot_general(h, fc2w[...], (((1,), (1,)), ((), ())),
                            preferred_element_type=jnp.float32,
                            precision=_PREC) + fc2b[0]
    o_ref[0] = x


def _block_kernel_plain(lo, hi, x_ref, *refs):
    _block_compute(lo, hi, x_ref, *refs)


def _block_compute2(lo, hi, x_ref, ln1w, ln1b, qkvw, qkvb, projw, projb,
                    ln2w, ln2b, fc1w, fc1b, fc2w, fc2b, o_ref):
    x = x_ref[...].reshape(2 * NPAD, EMBED)
    h = _ln(x, ln1w[0], ln1b[0], 1e-6)
    qkv = lax.dot_general(h, qkvw[...], (((1,), (1,)), ((), ())),
                          preferred_element_type=jnp.float32,
                          precision=_PREC) + qkvb[0]
    scale = HDIM ** -0.5
    kio = lax.broadcasted_iota(jnp.int32, (NPAD, NPAD), 1)
    kmask = (kio >= lo) & (kio < hi)
    atts = []
    for s in range(2):
        base = s * NPAD
        ss = []
        for hh in range(HEADS):
            q = qkv[base:base + NPAD, hh * HDIM:(hh + 1) * HDIM]
            k = qkv[base:base + NPAD, EMBED + hh * HDIM:EMBED + (hh + 1) * HDIM]
            ss.append(lax.dot_general(q * scale, k, (((1,), (1,)), ((), ())),
                                      preferred_element_type=jnp.float32,
                                      precision=_PREC))
        ps = []
        for hh in range(HEADS):
            e = jnp.where(kmask, jnp.exp(ss[hh]), 0.0)
            ps.append(e * (1.0 / jnp.sum(e, axis=-1, keepdims=True)))
        outs = []
        for hh in range(HEADS):
            v = qkv[base:base + NPAD,
                    2 * EMBED + hh * HDIM:2 * EMBED + (hh + 1) * HDIM]
            outs.append(lax.dot_general(ps[hh], v, (((1,), (0,)), ((), ())),
                                        preferred_element_type=jnp.float32,
                                        precision=_PREC))
        atts.append(jnp.concatenate(outs, axis=1))
    att = jnp.concatenate(atts, axis=0)
    x = x + lax.dot_general(att, projw[...], (((1,), (1,)), ((), ())),
                            preferred_element_type=jnp.float32,
                            precision=_PREC) + projb[0]
    h = _ln(x, ln2w[0], ln2b[0], 1e-6)
    h = _gelu(lax.dot_general(h, fc1w[...], (((1,), (1,)), ((), ())),
                              preferred_element_type=jnp.float32,
                              precision=_PREC) + fc1b[0])
    x = x + lax.dot_general(h, fc2w[...], (((1,), (1,)), ((), ())),
                            preferred_element_type=jnp.float32,
                            precision=_PREC) + fc2b[0]
    o_ref[...] = x.reshape(2, NPAD, EMBED)


def _block_kernel_gated(lo, hi, active_ref, x_ref, *refs):
    b = pl.program_id(0)
    o_ref = refs[-1]

    @pl.when(active_ref[b] != 0)
    def _():
        _block_compute(lo, hi, x_ref, *refs)

    @pl.when(active_ref[b] == 0)
    def _():
        o_ref[0] = x_ref[0]


def _bcast8(v):
    return jnp.broadcast_to(v[None, :], (8, v.shape[0]))


def _run_block(xx, blk, lo, hi, active=None):
    bsz = xx.shape[0]
    wparams = [
        _bcast8(blk['ln1_w']), _bcast8(blk['ln1_b']),
        blk['qkv_w'], _bcast8(blk['qkv_b']),
        blk['proj_w'], _bcast8(blk['proj_b']),
        _bcast8(blk['ln2_w']), _bcast8(blk['ln2_b']),
        blk['fc1_w'], _bcast8(blk['fc1_b']),
        blk['fc2_w'], _bcast8(blk['fc2_b']),
    ]

    def wspec(arr):
        nd = arr.ndim
        return pl.BlockSpec(arr.shape, lambda b, _n=nd: (0,) * _n)

    x_spec = pl.BlockSpec((1, NPAD, EMBED), lambda b: (b, 0, 0))
    w_specs = [wspec(a) for a in wparams]
    out_spec = pl.BlockSpec((1, NPAD, EMBED), lambda b: (b, 0, 0))
    out_shape = jax.ShapeDtypeStruct((bsz, NPAD, EMBED), jnp.float32)

    if active is None:
        x2_spec = pl.BlockSpec((2, NPAD, EMBED), lambda b: (b, 0, 0))
        return pl.pallas_call(
            functools.partial(_block_compute2, lo, hi),
            grid=(bsz // 2,),
            in_specs=[x2_spec] + w_specs,
            out_specs=x2_spec,
            out_shape=out_shape,
        )(xx, *wparams)
    act_spec = pl.BlockSpec(memory_space=pltpu.SMEM)
    return pl.pallas_call(
        functools.partial(_block_kernel_gated, lo, hi),
        grid=(bsz,),
        in_specs=[act_spec, x_spec] + w_specs,
        out_specs=out_spec,
        out_shape=out_shape,
    )(active, xx, *wparams)


def _patch_kernel(p_ref, w_ref, b_ref, o_ref):
    o_ref[...] = lax.dot_general(
        p_ref[...], w_ref[...], (((1,), (1,)), ((), ())),
        preferred_element_type=jnp.float32, precision=_PREC) + b_ref[0]


def _patch_embed(x, patch_w, patch_b):
    bsz = x.shape[0]
    g = IMG // PATCH
    pch = x.reshape(bsz, 3, g, PATCH, g, PATCH)
    pch = pch.transpose(0, 2, 4, 1, 3, 5).reshape(bsz * NPATCH, 3 * PATCH * PATCH)
    wm = patch_w.reshape(EMBED, -1)
    bb = _bcast8(patch_b)
    rows = bsz * NPATCH
    return pl.pallas_call(
        _patch_kernel,
        grid=(1,),
        in_specs=[
            pl.BlockSpec((rows, 3 * PATCH * PATCH), lambda i: (0, 0)),
            pl.BlockSpec(wm.shape, lambda i: (0, 0)),
            pl.BlockSpec(bb.shape, lambda i: (0, 0)),
        ],
        out_specs=pl.BlockSpec((rows, EMBED), lambda i: (0, 0)),
        out_shape=jax.ShapeDtypeStruct((rows, EMBED), jnp.float32),
    )(pch, wm, bb).reshape(bsz, NPATCH, EMBED)


def _gate_kernel_sc(logits_hbm, budget_hbm, out_hbm, lg_v, bud_v, act_v):
    wid = lax.axis_index("s") * 2 + lax.axis_index("c")

    @pl.when(wid == 0)
    def _():
        pltpu.sync_copy(logits_hbm, lg_v)
        pltpu.sync_copy(budget_hbm, bud_v)
        b6 = bud_v[...] * float(NUM_ADAPT)
        t = b6.astype(jnp.int32)
        k = t + jnp.where(t.astype(jnp.float32) < b6, 1, 0)
        k = jnp.maximum(k, 1)
        ls = [lg_v[j] for j in range(NUM_ADAPT)]
        one = jnp.full((16,), 1, jnp.int32)
        zero = jnp.full((16,), 0, jnp.int32)
        for j in range(NUM_ADAPT):
            r = zero
            for j2 in range(NUM_ADAPT):
                if j2 == j:
                    continue
                beats = ls[j2] > ls[j]
                if j2 < j:
                    beats = beats | (ls[j2] == ls[j])
                r = r + jnp.where(beats, one, zero)
            act_v[j] = jnp.where(r < k, one, zero)
        pltpu.sync_copy(act_v, out_hbm)


def _gate_sc(logits, budget):
    mesh = plsc.VectorSubcoreMesh(core_axis_name="c", subcore_axis_name="s")
    bsz = budget.shape[0]
    f = functools.partial(
        pl.kernel, mesh=mesh,
        out_type=jax.ShapeDtypeStruct((NUM_ADAPT, bsz), jnp.int32),
        scratch_types=[
            pltpu.VMEM((NUM_ADAPT, bsz), jnp.float32),
            pltpu.VMEM((bsz,), jnp.float32),
            pltpu.VMEM((NUM_ADAPT, bsz), jnp.int32),
        ],
    )(_gate_kernel_sc)
    return f(logits.T, budget)


def kernel(x, budget, params, lat_table):
    bsz = x.shape[0]
    b2 = budget[:, None]

    scaled = b2 * (MAX_BUDGET - 1)
    idx = jnp.round(scaled).astype(jnp.int32).squeeze(-1)
    enc = jnp.take(lat_table, idx, axis=0)
    h = enc @ params['lat_fc1_w'].T + params['lat_fc1_b']
    h = _gelu(h)
    h = _ln(h, params['lat_ln_w'], params['lat_ln_b'], 1e-5)
    lat_tok = (h @ params['lat_fc2_w'].T + params['lat_fc2_b'])[:, None, :]

    tokens = _patch_embed(x, params['patch_w'], params['patch_b'])

    cls = jnp.broadcast_to(params['cls_token'], (bsz, 1, EMBED))
    n_fixed_valid = NPATCH + 2
    xx = jnp.concatenate(
        [jnp.concatenate([lat_tok, cls, tokens], axis=1) + params['pos_embed'],
         jnp.zeros((bsz, NPAD - n_fixed_valid, EMBED), jnp.float32)], axis=1)

    for i in range(NUM_FIXED):
        xx = _run_block(xx, params['blocks'][i], 0, n_fixed_valid)

    lat_repr = xx[:, 0]
    logits = lat_repr @ params['sched_w'].T + params['sched_b']
    active_t = _gate_sc(logits, budget)

    toks = xx
    for i in range(NUM_ADAPT):
        blk = params['blocks'][NUM_FIXED + i]
        toks = _run_block(toks, blk, 1, n_fixed_valid, active=active_t[i])

    cls_out = _ln(toks[:, 1], params['norm_w'], params['norm_b'], 1e-6)
    return cls_out @ params['head_w'].T + params['head_b']
